# baseline (speedup 1.0000x reference)
import functools

import jax
import jax.numpy as jnp
from jax import lax
from jax.experimental import pallas as pl
from jax.experimental.pallas import tpu as pltpu
from jax.experimental.pallas import tpu_sc as plsc

N = 10000
D = 128
E = 320000
FF = 4 * D
NC = 2
NS = 16
NW = NC * NS
NPAD = 10240
EPAD = 327680
CH = 128
EPTD = EPAD // NW
DCHUNK = EPTD // CH
EPT = EPAD // NS
NCHUNK = EPT // CH
ROWS_PT = NPAD // NS
DH = D // 2
RB = 1024
G = NPAD // RB

@functools.cache
def _get_mesh():
    return plsc.VectorSubcoreMesh(
        core_axis_name="c", subcore_axis_name="s",
        num_cores=NC, num_subcores=NS)


def _deg_body(dst_hbm, deg_out, dstv, onesv, stage, deg_sp):
    c = lax.axis_index("c")
    s = lax.axis_index("s")
    wid = c * NS + s

    @pl.loop(0, ROWS_PT // 16)
    def _z(i):
        stage[pl.ds(i * 16, 16)] = jnp.zeros((16,), jnp.float32)

    @pl.loop(0, CH // 16)
    def _o(i):
        onesv[pl.ds(i * 16, 16)] = jnp.ones((16,), jnp.float32)

    pltpu.sync_copy(stage, deg_sp.at[pl.ds(s * ROWS_PT, ROWS_PT)])
    pltpu.sync_copy(dst_hbm.at[wid], dstv)
    plsc.subcore_barrier()

    @pl.loop(0, DCHUNK)
    def _hist(j):
        pltpu.sync_copy(onesv, deg_sp.at[dstv.at[j]], add=True)

    plsc.subcore_barrier()
    pltpu.sync_copy(deg_sp.at[pl.ds(s * ROWS_PT, ROWS_PT)], stage)
    pltpu.sync_copy(stage, deg_out.at[pl.ds(c * NPAD + s * ROWS_PT, ROWS_PT)])


@functools.cache
def _deg():
    return pl.kernel(
        _deg_body,
        out_type=jax.ShapeDtypeStruct((NC * NPAD,), jnp.float32),
        mesh=_get_mesh(),
        scratch_types=[
            pltpu.VMEM((DCHUNK, CH), jnp.int32),
            pltpu.VMEM((CH,), jnp.float32),
            pltpu.VMEM((ROWS_PT,), jnp.float32),
            pltpu.VMEM_SHARED((NPAD,), jnp.float32),
        ],
    )


NB = 4


def _agg_body(hws_hbm, src_hbm, dst_hbm, out, srcv, dstv, bufs, gsems, ssems,
              acc_sp):
    c = lax.axis_index("c")
    s = lax.axis_index("s")
    buf0 = bufs[0]

    @pl.loop(0, CH)
    def _z(r):
        for k in range(DH // 16):
            buf0[r, pl.ds(k * 16, 16)] = jnp.zeros((16,), jnp.float32)

    for t in range(ROWS_PT // CH):
        pltpu.sync_copy(buf0, acc_sp.at[pl.ds(s * ROWS_PT + t * CH, CH)])

    pltpu.sync_copy(src_hbm.at[c, s], srcv)
    pltpu.sync_copy(dst_hbm.at[s], dstv)
    plsc.subcore_barrier()

    def _issue(j, buf, sem):
        pltpu.async_copy(hws_hbm.at[srcv.at[j]], buf, sem)

    def _drain(buf, sem):
        pltpu.make_async_copy(hws_hbm.at[pl.ds(0, CH)], buf, sem).wait()

    for b in range(NB):
        _issue(b, bufs[b], gsems[b])

    NG = NCHUNK // NB

    @pl.loop(0, NG)
    def _body(i):
        descs = []
        for b in range(NB):
            j = i * NB + b
            _drain(bufs[b], gsems[b])
            descs.append(pltpu.async_copy(
                bufs[b], acc_sp.at[dstv.at[j]], ssems[b], add=True))
        for d in descs:
            d.wait()

        @pl.when(i < NG - 1)
        def _n():
            for b in range(NB):
                _issue((i + 1) * NB + b, bufs[b], gsems[b])

    plsc.subcore_barrier()
    for t in range(ROWS_PT // CH):
        row0 = s * ROWS_PT + t * CH
        pltpu.sync_copy(acc_sp.at[pl.ds(row0, CH)], buf0)
        pltpu.sync_copy(buf0, out.at[pl.ds(c * NPAD + row0, CH)])


@functools.cache
def _agg():
    return pl.kernel(
        _agg_body,
        out_type=jax.ShapeDtypeStruct((NC * NPAD, DH), jnp.float32),
        mesh=_get_mesh(),
        scratch_types=[
            pltpu.VMEM((NCHUNK, CH), jnp.int32),
            pltpu.VMEM((NCHUNK, CH), jnp.int32),
            [pltpu.VMEM((CH, DH), jnp.float32) for _ in range(NB)],
            [pltpu.SemaphoreType.DMA for _ in range(NB)],
            [pltpu.SemaphoreType.DMA for _ in range(NB)],
            pltpu.VMEM_SHARED((NPAD, DH), jnp.float32),
        ],
        compiler_params=pltpu.CompilerParams(use_tc_tiling_on_sc=False),
    )


def _hws_body(x_ref, w_ref, degp_ref, hws_ref):
    dp = degp_ref[...]
    deg = dp[0, 0, :] + dp[0, 1, :] + 1.0
    dinv = lax.rsqrt(deg).reshape(RB, 1)
    hw = jnp.dot(x_ref[...], w_ref[...], preferred_element_type=jnp.float32)
    hws_ref[...] = hw * dinv


def _hws_call(x_pad, W, degp3):
    return pl.pallas_call(
        _hws_body,
        grid=(G,),
        in_specs=[
            pl.BlockSpec((RB, D), lambda g: (g, 0)),
            pl.BlockSpec((D, D), lambda g: (0, 0)),
            pl.BlockSpec((1, NC, RB), lambda g: (g, 0, 0)),
        ],
        out_specs=pl.BlockSpec((RB, D), lambda g: (g, 0)),
        out_shape=jax.ShapeDtypeStruct((NPAD, D), jnp.float32),
    )(x_pad, W, degp3)


def _ln(h, g, b):
    m = jnp.mean(h, axis=1, keepdims=True)
    v = jnp.mean((h - m) * (h - m), axis=1, keepdims=True)
    return (h - m) * lax.rsqrt(v + 1e-5) * g + b


def _tail_body(p_ref, hws_ref, x_ref, degp_ref, b_ref, w1_ref, b1_ref,
               w2_ref, b2_ref, g1_ref, be1_ref, g2_ref, be2_ref, o_ref):
    dp = degp_ref[...]
    deg = dp[0, 0, :] + dp[0, 1, :] + 1.0
    dinv = lax.rsqrt(deg).reshape(RB, 1)
    hws = hws_ref[...]
    p = p_ref[...]
    pcat = jnp.concatenate([p[0], p[1]], axis=1)
    agg = (pcat + hws) * dinv + b_ref[...]
    h1 = agg + x_ref[...]
    h2 = _ln(h1, g1_ref[...], be1_ref[...])
    a = jnp.dot(h2, w1_ref[...], preferred_element_type=jnp.float32) + b1_ref[...]
    a = a * jax.nn.sigmoid(a)
    ff = jnp.dot(a, w2_ref[...], preferred_element_type=jnp.float32) + b2_ref[...]
    o_ref[...] = _ln(ff + h2, g2_ref[...], be2_ref[...])


def _tail_call(p, hws, x_pad, degp3, b, W1, b1, W2, b2, g1, be1, g2, be2):
    vec = lambda v: v.reshape(1, -1)
    full = lambda shape: pl.BlockSpec(shape, lambda g: tuple(0 for _ in shape))
    return pl.pallas_call(
        _tail_body,
        grid=(G,),
        in_specs=[
            pl.BlockSpec((NC, RB, DH), lambda g: (0, g, 0)),
            pl.BlockSpec((RB, D), lambda g: (g, 0)),
            pl.BlockSpec((RB, D), lambda g: (g, 0)),
            pl.BlockSpec((1, NC, RB), lambda g: (g, 0, 0)),
            full((1, D)), full((D, FF)), full((1, FF)),
            full((FF, D)), full((1, D)),
            full((1, D)), full((1, D)), full((1, D)), full((1, D)),
        ],
        out_specs=pl.BlockSpec((RB, D), lambda g: (g, 0)),
        out_shape=jax.ShapeDtypeStruct((NPAD, D), jnp.float32),
    )(p, hws, x_pad, degp3, vec(b), W1, vec(b1), W2, vec(b2),
      vec(g1), vec(be1), vec(g2), vec(be2))


def kernel(x, edge_index, W, b, W1, b1, W2, b2, g1, be1, g2, be2):
    x2 = x.reshape(N, D)
    x_pad = jnp.pad(x2, ((0, NPAD - N), (0, 0)))

    pad = EPAD - E
    src = jnp.concatenate([edge_index[0], jnp.zeros((pad,), jnp.int32)])
    dst = jnp.concatenate([edge_index[1], jnp.full((pad,), N, jnp.int32)])
    dst_deg = dst.reshape(NW, DCHUNK, CH)
    src_agg = jnp.stack([2 * src, 2 * src + 1]).reshape(NC, NS, NCHUNK, CH)
    dst_agg = dst.reshape(NS, NCHUNK, CH)

    degp = _deg()(dst_deg)
    degp3 = degp.reshape(NC, G, RB).swapaxes(0, 1)

    hws = _hws_call(x_pad, W, degp3)
    hws2 = hws.reshape(2 * NPAD, DH)

    p = _agg()(hws2, src_agg, dst_agg).reshape(NC, NPAD, DH)

    out = _tail_call(p, hws, x_pad, degp3,
                     b, W1, b1, W2, b2, g1, be1, g2, be2)
    return out[:N].reshape(1, N, D)


# speedup vs baseline: 1.8261x; 1.8261x over previous
import functools

import jax
import jax.numpy as jnp
from jax import lax
from jax.experimental import pallas as pl
from jax.experimental.pallas import tpu as pltpu
from jax.experimental.pallas import tpu_sc as plsc

N = 10000
D = 128
E = 320000
FF = 4 * D
NC = 2
NS = 16
NW = NC * NS
NPAD = 10240
EPAD = 327680
CH = 128
EPTD = EPAD // NW
DCHUNK = EPTD // CH
EPT = EPAD // NS
NCHUNK = EPT // CH
ROWS_PT = NPAD // NS
DH = D // 2
RB = 1024
G = NPAD // RB

@functools.cache
def _get_mesh():
    return plsc.VectorSubcoreMesh(
        core_axis_name="c", subcore_axis_name="s",
        num_cores=NC, num_subcores=NS)


def _deg_body(dst_hbm, deg_out, dstv, onesv, stage, deg_sp):
    c = lax.axis_index("c")
    s = lax.axis_index("s")
    wid = c * NS + s

    @pl.loop(0, ROWS_PT // 16)
    def _z(i):
        stage[pl.ds(i * 16, 16)] = jnp.zeros((16,), jnp.float32)

    @pl.loop(0, CH // 16)
    def _o(i):
        onesv[pl.ds(i * 16, 16)] = jnp.ones((16,), jnp.float32)

    pltpu.sync_copy(stage, deg_sp.at[pl.ds(s * ROWS_PT, ROWS_PT)])
    pltpu.sync_copy(dst_hbm.at[wid], dstv)
    plsc.subcore_barrier()

    @pl.loop(0, DCHUNK)
    def _hist(j):
        pltpu.sync_copy(onesv, deg_sp.at[dstv.at[j]], add=True)

    plsc.subcore_barrier()
    pltpu.sync_copy(deg_sp.at[pl.ds(s * ROWS_PT, ROWS_PT)], stage)
    pltpu.sync_copy(stage, deg_out.at[pl.ds(c * NPAD + s * ROWS_PT, ROWS_PT)])


@functools.cache
def _deg():
    return pl.kernel(
        _deg_body,
        out_type=jax.ShapeDtypeStruct((NC * NPAD,), jnp.float32),
        mesh=_get_mesh(),
        scratch_types=[
            pltpu.VMEM((DCHUNK, CH), jnp.int32),
            pltpu.VMEM((CH,), jnp.float32),
            pltpu.VMEM((ROWS_PT,), jnp.float32),
            pltpu.VMEM_SHARED((NPAD,), jnp.float32),
        ],
    )


GK = 8
NGRP = NCHUNK // GK
ZR = 128


def _agg_body(hws_hbm, src_hbm, dst_hbm, out, sring, dring, dummy_idx,
              bufs, ssems, table_sp, acc_sp):
    c = lax.axis_index("c")
    s = lax.axis_index("s")
    buf0 = bufs[0]

    @pl.loop(0, ZR)
    def _z(r):
        for k in range(DH // 16):
            buf0[r, pl.ds(k * 16, 16)] = jnp.zeros((16,), jnp.float32)

    for t in range(ROWS_PT // ZR):
        pltpu.sync_copy(buf0, acc_sp.at[pl.ds(s * ROWS_PT + t * ZR, ZR)])

    @pl.loop(0, CH // 16)
    def _di(k):
        dummy_idx[0, pl.ds(k * 16, 16)] = jnp.full((16,), N, jnp.int32)

    for t in range(ROWS_PT // CH):
        row0 = s * ROWS_PT + t * CH
        pltpu.sync_copy(hws_hbm.at[c, pl.ds(row0, CH)], buf0)
        pltpu.sync_copy(buf0, table_sp.at[pl.ds(row0, CH)])

    plsc.subcore_barrier()

    def _drain_scatter(buf, sem):
        pltpu.make_async_copy(buf, acc_sp.at[dummy_idx.at[0]], sem).wait()

    for b in range(2):
        pltpu.async_copy(bufs[b], acc_sp.at[dummy_idx.at[0]], ssems[b],
                         add=True)

    @pl.loop(0, NGRP)
    def _grp(g):
        pltpu.sync_copy(src_hbm.at[s, pl.ds(g * GK, GK)], sring)
        pltpu.sync_copy(dst_hbm.at[s, pl.ds(g * GK, GK)], dring)
        for kc in range(GK):
            buf = bufs[kc & 1]
            sem = ssems[kc & 1]
            _drain_scatter(buf, sem)
            pltpu.sync_copy(table_sp.at[sring.at[kc]], buf)
            pltpu.async_copy(buf, acc_sp.at[dring.at[kc]], sem, add=True)

    for b in range(2):
        _drain_scatter(bufs[b], ssems[b])

    plsc.subcore_barrier()
    for t in range(ROWS_PT // CH):
        row0 = s * ROWS_PT + t * CH
        pltpu.sync_copy(acc_sp.at[pl.ds(row0, CH)], buf0)
        pltpu.sync_copy(buf0, out.at[pl.ds(c * NPAD + row0, CH)])


@functools.cache
def _agg():
    return pl.kernel(
        _agg_body,
        out_type=jax.ShapeDtypeStruct((NC * NPAD, DH), jnp.float32),
        mesh=_get_mesh(),
        scratch_types=[
            pltpu.VMEM((GK, CH), jnp.int32),
            pltpu.VMEM((GK, CH), jnp.int32),
            pltpu.VMEM((1, CH), jnp.int32),
            [pltpu.VMEM((CH, DH), jnp.float32) for _ in range(2)],
            [pltpu.SemaphoreType.DMA for _ in range(2)],
            pltpu.VMEM_SHARED((NPAD, DH), jnp.float32),
            pltpu.VMEM_SHARED((NPAD, DH), jnp.float32),
        ],
        compiler_params=pltpu.CompilerParams(use_tc_tiling_on_sc=False),
    )


def _hws_body(x_ref, w_ref, degp_ref, hws_ref, hsp_ref):
    dp = degp_ref[...]
    deg = dp[0, 0, :] + dp[0, 1, :] + 1.0
    dinv = lax.rsqrt(deg).reshape(RB, 1)
    hw = jnp.dot(x_ref[...], w_ref[...], preferred_element_type=jnp.float32)
    hws = hw * dinv
    hws_ref[...] = hws
    hsp_ref[0] = hws[:, :DH]
    hsp_ref[1] = hws[:, DH:]


def _hws_call(x_pad, W, degp3):
    return pl.pallas_call(
        _hws_body,
        grid=(G,),
        in_specs=[
            pl.BlockSpec((RB, D), lambda g: (g, 0)),
            pl.BlockSpec((D, D), lambda g: (0, 0)),
            pl.BlockSpec((1, NC, RB), lambda g: (g, 0, 0)),
        ],
        out_specs=[
            pl.BlockSpec((RB, D), lambda g: (g, 0)),
            pl.BlockSpec((NC, RB, DH), lambda g: (0, g, 0)),
        ],
        out_shape=[
            jax.ShapeDtypeStruct((NPAD, D), jnp.float32),
            jax.ShapeDtypeStruct((NC, NPAD, DH), jnp.float32),
        ],
    )(x_pad, W, degp3)


def _ln(h, g, b):
    m = jnp.mean(h, axis=1, keepdims=True)
    v = jnp.mean((h - m) * (h - m), axis=1, keepdims=True)
    return (h - m) * lax.rsqrt(v + 1e-5) * g + b


def _tail_body(p_ref, hws_ref, x_ref, degp_ref, b_ref, w1_ref, b1_ref,
               w2_ref, b2_ref, g1_ref, be1_ref, g2_ref, be2_ref, o_ref):
    dp = degp_ref[...]
    deg = dp[0, 0, :] + dp[0, 1, :] + 1.0
    dinv = lax.rsqrt(deg).reshape(RB, 1)
    hws = hws_ref[...]
    p = p_ref[...]
    pcat = jnp.concatenate([p[0], p[1]], axis=1)
    agg = (pcat + hws) * dinv + b_ref[...]
    h1 = agg + x_ref[...]
    h2 = _ln(h1, g1_ref[...], be1_ref[...])
    a = jnp.dot(h2, w1_ref[...], preferred_element_type=jnp.float32) + b1_ref[...]
    a = a * jax.nn.sigmoid(a)
    ff = jnp.dot(a, w2_ref[...], preferred_element_type=jnp.float32) + b2_ref[...]
    o_ref[...] = _ln(ff + h2, g2_ref[...], be2_ref[...])


def _tail_call(p, hws, x_pad, degp3, b, W1, b1, W2, b2, g1, be1, g2, be2):
    vec = lambda v: v.reshape(1, -1)
    full = lambda shape: pl.BlockSpec(shape, lambda g: tuple(0 for _ in shape))
    return pl.pallas_call(
        _tail_body,
        grid=(G,),
        in_specs=[
            pl.BlockSpec((NC, RB, DH), lambda g: (0, g, 0)),
            pl.BlockSpec((RB, D), lambda g: (g, 0)),
            pl.BlockSpec((RB, D), lambda g: (g, 0)),
            pl.BlockSpec((1, NC, RB), lambda g: (g, 0, 0)),
            full((1, D)), full((D, FF)), full((1, FF)),
            full((FF, D)), full((1, D)),
            full((1, D)), full((1, D)), full((1, D)), full((1, D)),
        ],
        out_specs=pl.BlockSpec((RB, D), lambda g: (g, 0)),
        out_shape=jax.ShapeDtypeStruct((NPAD, D), jnp.float32),
    )(p, hws, x_pad, degp3, vec(b), W1, vec(b1), W2, vec(b2),
      vec(g1), vec(be1), vec(g2), vec(be2))


def kernel(x, edge_index, W, b, W1, b1, W2, b2, g1, be1, g2, be2):
    x2 = x.reshape(N, D)
    x_pad = jnp.pad(x2, ((0, NPAD - N), (0, 0)))

    pad = EPAD - E
    src = jnp.concatenate([edge_index[0], jnp.zeros((pad,), jnp.int32)])
    dst = jnp.concatenate([edge_index[1], jnp.full((pad,), N, jnp.int32)])
    dst_deg = dst.reshape(NW, DCHUNK, CH)
    src_agg = src.reshape(NS, NCHUNK, CH)
    dst_agg = dst.reshape(NS, NCHUNK, CH)

    degp = _deg()(dst_deg)
    degp3 = degp.reshape(NC, G, RB).swapaxes(0, 1)

    hws, hws_split = _hws_call(x_pad, W, degp3)

    p = _agg()(hws_split, src_agg, dst_agg).reshape(NC, NPAD, DH)

    out = _tail_call(p, hws, x_pad, degp3,
                     b, W1, b1, W2, b2, g1, be1, g2, be2)
    return out[:N].reshape(1, N, D)


# speedup vs baseline: 1.8918x; 1.0360x over previous
import functools

import jax
import jax.numpy as jnp
from jax import lax
from jax.experimental import pallas as pl
from jax.experimental.pallas import tpu as pltpu
from jax.experimental.pallas import tpu_sc as plsc

N = 10000
D = 128
E = 320000
FF = 4 * D
NC = 2
NS = 16
NW = NC * NS
NPAD = 10240
EPAD = 327680
CH = 128
EPTD = EPAD // NW
DCHUNK = EPTD // CH
EPT = EPAD // NS
NCHUNK = EPT // CH
ROWS_PT = NPAD // NS
DH = D // 2
RB = 1024
G = NPAD // RB

@functools.cache
def _get_mesh():
    return plsc.VectorSubcoreMesh(
        core_axis_name="c", subcore_axis_name="s",
        num_cores=NC, num_subcores=NS)


def _deg_body(dst_hbm, deg_out, dstv, onesv, stage, deg_sp):
    c = lax.axis_index("c")
    s = lax.axis_index("s")
    wid = c * NS + s

    @pl.loop(0, ROWS_PT // 16)
    def _z(i):
        stage[pl.ds(i * 16, 16)] = jnp.zeros((16,), jnp.float32)

    @pl.loop(0, CH // 16)
    def _o(i):
        onesv[pl.ds(i * 16, 16)] = jnp.ones((16,), jnp.float32)

    pltpu.sync_copy(stage, deg_sp.at[pl.ds(s * ROWS_PT, ROWS_PT)])
    pltpu.sync_copy(dst_hbm.at[wid], dstv)
    plsc.subcore_barrier()

    @pl.loop(0, DCHUNK)
    def _hist(j):
        pltpu.sync_copy(onesv, deg_sp.at[dstv.at[j]], add=True)

    plsc.subcore_barrier()
    pltpu.sync_copy(deg_sp.at[pl.ds(s * ROWS_PT, ROWS_PT)], stage)
    pltpu.sync_copy(stage, deg_out.at[pl.ds(c * NPAD + s * ROWS_PT, ROWS_PT)])


@functools.cache
def _deg():
    return pl.kernel(
        _deg_body,
        out_type=jax.ShapeDtypeStruct((NC * NPAD,), jnp.float32),
        mesh=_get_mesh(),
        scratch_types=[
            pltpu.VMEM((DCHUNK, CH), jnp.int32),
            pltpu.VMEM((CH,), jnp.float32),
            pltpu.VMEM((ROWS_PT,), jnp.float32),
            pltpu.VMEM_SHARED((NPAD,), jnp.float32),
        ],
    )


GK = 16
NGRP = NCHUNK // GK
ZR = 128
NBUF = 3


def _agg_body(hws_hbm, src_hbm, dst_hbm, out, sring, dring, dummy_idx,
              bufs, ssems, gsems, table_sp, acc_sp):
    c = lax.axis_index("c")
    s = lax.axis_index("s")
    buf0 = bufs[0]

    @pl.loop(0, ZR)
    def _z(r):
        for k in range(DH // 16):
            buf0[r, pl.ds(k * 16, 16)] = jnp.zeros((16,), jnp.float32)

    for t in range(ROWS_PT // ZR):
        pltpu.sync_copy(buf0, acc_sp.at[pl.ds(s * ROWS_PT + t * ZR, ZR)])

    @pl.loop(0, CH // 16)
    def _di(k):
        dummy_idx[0, pl.ds(k * 16, 16)] = jnp.full((16,), N, jnp.int32)

    for t in range(ROWS_PT // CH):
        row0 = s * ROWS_PT + t * CH
        pltpu.sync_copy(hws_hbm.at[c, pl.ds(row0, CH)], buf0)
        pltpu.sync_copy(buf0, table_sp.at[pl.ds(row0, CH)])

    plsc.subcore_barrier()

    def _drain_scatter(b):
        pltpu.make_async_copy(bufs[b], acc_sp.at[dummy_idx.at[0]],
                              ssems[b]).wait()

    def _drain_gather(b):
        pltpu.make_async_copy(table_sp.at[pl.ds(0, CH)], bufs[b],
                              gsems[b]).wait()

    def _issue_gather(kc, b):
        pltpu.async_copy(table_sp.at[sring.at[kc]], bufs[b], gsems[b])

    for b in range(NBUF):
        pltpu.async_copy(bufs[b], acc_sp.at[dummy_idx.at[0]], ssems[b],
                         add=True)

    @pl.loop(0, NGRP)
    def _grp(g):
        pltpu.sync_copy(src_hbm.at[s, pl.ds(g * GK, GK)], sring)
        pltpu.sync_copy(dst_hbm.at[s, pl.ds(g * GK, GK)], dring)
        _drain_scatter(0)
        _issue_gather(0, 0)
        _drain_scatter(1)
        _issue_gather(1, 1)
        for kc in range(GK):
            b = kc % NBUF
            if kc + 2 < GK:
                b2 = (kc + 2) % NBUF
                _drain_scatter(b2)
                _issue_gather(kc + 2, b2)
            _drain_gather(b)
            pltpu.async_copy(bufs[b], acc_sp.at[dring.at[kc]], ssems[b],
                             add=True)

    for b in range(NBUF):
        _drain_scatter(b)

    plsc.subcore_barrier()
    for t in range(ROWS_PT // CH):
        row0 = s * ROWS_PT + t * CH
        pltpu.sync_copy(acc_sp.at[pl.ds(row0, CH)], buf0)
        pltpu.sync_copy(buf0, out.at[pl.ds(c * NPAD + row0, CH)])


@functools.cache
def _agg():
    return pl.kernel(
        _agg_body,
        out_type=jax.ShapeDtypeStruct((NC * NPAD, DH), jnp.float32),
        mesh=_get_mesh(),
        scratch_types=[
            pltpu.VMEM((GK, CH), jnp.int32),
            pltpu.VMEM((GK, CH), jnp.int32),
            pltpu.VMEM((1, CH), jnp.int32),
            [pltpu.VMEM((CH, DH), jnp.float32) for _ in range(NBUF)],
            [pltpu.SemaphoreType.DMA for _ in range(NBUF)],
            [pltpu.SemaphoreType.DMA for _ in range(NBUF)],
            pltpu.VMEM_SHARED((NPAD, DH), jnp.float32),
            pltpu.VMEM_SHARED((NPAD, DH), jnp.float32),
        ],
        compiler_params=pltpu.CompilerParams(use_tc_tiling_on_sc=False),
    )


def _hws_body(x_ref, w_ref, degp_ref, hws_ref, hsp_ref):
    dp = degp_ref[...]
    deg = dp[0, 0, :] + dp[0, 1, :] + 1.0
    dinv = lax.rsqrt(deg).reshape(RB, 1)
    hw = jnp.dot(x_ref[...], w_ref[...], preferred_element_type=jnp.float32)
    hws = hw * dinv
    hws_ref[...] = hws
    hsp_ref[0] = hws[:, :DH]
    hsp_ref[1] = hws[:, DH:]


def _hws_call(x2, W, degp3):
    return pl.pallas_call(
        _hws_body,
        grid=(G,),
        in_specs=[
            pl.BlockSpec((RB, D), lambda g: (g, 0)),
            pl.BlockSpec((D, D), lambda g: (0, 0)),
            pl.BlockSpec((1, NC, RB), lambda g: (g, 0, 0)),
        ],
        out_specs=[
            pl.BlockSpec((RB, D), lambda g: (g, 0)),
            pl.BlockSpec((NC, RB, DH), lambda g: (0, g, 0)),
        ],
        out_shape=[
            jax.ShapeDtypeStruct((NPAD, D), jnp.float32),
            jax.ShapeDtypeStruct((NC, NPAD, DH), jnp.float32),
        ],
    )(x2, W, degp3)


def _ln(h, g, b):
    m = jnp.mean(h, axis=1, keepdims=True)
    v = jnp.mean((h - m) * (h - m), axis=1, keepdims=True)
    return (h - m) * lax.rsqrt(v + 1e-5) * g + b


def _tail_body(p_ref, hws_ref, x_ref, degp_ref, b_ref, w1_ref, b1_ref,
               w2_ref, b2_ref, g1_ref, be1_ref, g2_ref, be2_ref, o_ref):
    dp = degp_ref[...]
    deg = dp[0, 0, :] + dp[0, 1, :] + 1.0
    dinv = lax.rsqrt(deg).reshape(RB, 1)
    hws = hws_ref[...]
    p = p_ref[...]
    pcat = jnp.concatenate([p[0], p[1]], axis=1)
    agg = (pcat + hws) * dinv + b_ref[...]
    h1 = agg + x_ref[...]
    h2 = _ln(h1, g1_ref[...], be1_ref[...])
    a = jnp.dot(h2, w1_ref[...], preferred_element_type=jnp.float32) + b1_ref[...]
    a = a * jax.nn.sigmoid(a)
    ff = jnp.dot(a, w2_ref[...], preferred_element_type=jnp.float32) + b2_ref[...]
    o_ref[...] = _ln(ff + h2, g2_ref[...], be2_ref[...])


def _tail_call(p, hws, x2, degp3, b, W1, b1, W2, b2, g1, be1, g2, be2):
    vec = lambda v: v.reshape(1, -1)
    full = lambda shape: pl.BlockSpec(shape, lambda g: tuple(0 for _ in shape))
    return pl.pallas_call(
        _tail_body,
        grid=(G,),
        in_specs=[
            pl.BlockSpec((NC, RB, DH), lambda g: (0, g, 0)),
            pl.BlockSpec((RB, D), lambda g: (g, 0)),
            pl.BlockSpec((RB, D), lambda g: (g, 0)),
            pl.BlockSpec((1, NC, RB), lambda g: (g, 0, 0)),
            full((1, D)), full((D, FF)), full((1, FF)),
            full((FF, D)), full((1, D)),
            full((1, D)), full((1, D)), full((1, D)), full((1, D)),
        ],
        out_specs=pl.BlockSpec((RB, D), lambda g: (g, 0)),
        out_shape=jax.ShapeDtypeStruct((N, D), jnp.float32),
    )(p, hws, x2, degp3, vec(b), W1, vec(b1), W2, vec(b2),
      vec(g1), vec(be1), vec(g2), vec(be2))


def kernel(x, edge_index, W, b, W1, b1, W2, b2, g1, be1, g2, be2):
    x2 = x.reshape(N, D)

    pad = EPAD - E
    src = jnp.concatenate([edge_index[0], jnp.zeros((pad,), jnp.int32)])
    dst = jnp.concatenate([edge_index[1], jnp.full((pad,), N, jnp.int32)])
    dst_deg = dst.reshape(NW, DCHUNK, CH)
    src_agg = src.reshape(NS, NCHUNK, CH)
    dst_agg = dst.reshape(NS, NCHUNK, CH)

    degp = _deg()(dst_deg)
    degp3 = degp.reshape(NC, G, RB).swapaxes(0, 1)

    hws, hws_split = _hws_call(x2, W, degp3)

    p = _agg()(hws_split, src_agg, dst_agg).reshape(NC, NPAD, DH)

    out = _tail_call(p, hws, x2, degp3,
                     b, W1, b1, W2, b2, g1, be1, g2, be2)
    return out.reshape(1, N, D)
